# Initial kernel scaffold; baseline (speedup 1.0000x reference)
import jax
import jax.numpy as jnp
from jax.experimental import pallas as pl


def kernel(visit_order, pos_embed_weight):
    raise NotImplementedError("write your pallas kernel here")


# speedup vs baseline: 3.0498x; 3.0498x over previous
import functools

import jax
import jax.numpy as jnp
from jax import lax
from jax.experimental import pallas as pl
from jax.experimental.pallas import tpu as pltpu
from jax.experimental.pallas import tpu_sc as plsc

EMBED_NUM = 200
EMBED_DIM = 64
BATCH = 16384
HIST = 200

_B = BATCH * HIST
_IDX_MINOR = 128
_IDX_ROWS = _B // _IDX_MINOR

_NW = 32
_ROWS_PER_W = _IDX_ROWS // _NW
_ROWS_PER_STEP = 8
_CHUNK = _ROWS_PER_STEP * _IDX_MINOR
_STEPS = _ROWS_PER_W // _ROWS_PER_STEP


def _sc_gather(idx2d, table):
    mesh = plsc.VectorSubcoreMesh(core_axis_name="c", subcore_axis_name="s")

    @functools.partial(
        pl.kernel,
        mesh=mesh,
        out_type=jax.ShapeDtypeStruct((_B, EMBED_DIM), jnp.float32),
        scratch_types=[
            pltpu.VMEM((_ROWS_PER_STEP, _IDX_MINOR), jnp.int32),
            pltpu.VMEM((_CHUNK, EMBED_DIM), jnp.float32),
            pltpu.SemaphoreType.DMA,
        ],
        compiler_params=pltpu.CompilerParams(use_tc_tiling_on_sc=False),
    )
    def k(idx_hbm, table_hbm, out_hbm, idx_v, rows_v, sem):
        wid = lax.axis_index("s") * 2 + lax.axis_index("c")
        row0 = wid * _ROWS_PER_W

        def step(i, _):
            r = row0 + i * _ROWS_PER_STEP
            pltpu.sync_copy(idx_hbm.at[pl.ds(r, _ROWS_PER_STEP)], idx_v)
            for j in range(_ROWS_PER_STEP):
                pltpu.async_copy(
                    table_hbm.at[idx_v.at[j]],
                    rows_v.at[pl.ds(j * _IDX_MINOR, _IDX_MINOR)],
                    sem,
                )
            for j in range(_ROWS_PER_STEP):
                pltpu.make_async_copy(
                    table_hbm.at[idx_v.at[j]],
                    rows_v.at[pl.ds(j * _IDX_MINOR, _IDX_MINOR)],
                    sem,
                ).wait()
            pltpu.sync_copy(rows_v, out_hbm.at[pl.ds(r * _IDX_MINOR, _CHUNK)])
            return ()

        lax.fori_loop(0, _STEPS, step, (), unroll=False)

    return k(idx2d, table)


def kernel(visit_order, pos_embed_weight):
    idx2d = jnp.reshape(visit_order.astype(jnp.int32), (_IDX_ROWS, _IDX_MINOR))
    flat = _sc_gather(idx2d, pos_embed_weight)
    return jnp.reshape(flat, (BATCH, HIST, EMBED_DIM))


# speedup vs baseline: 5.2390x; 1.7178x over previous
import functools

import jax
import jax.numpy as jnp
from jax import lax
from jax.experimental import pallas as pl
from jax.experimental.pallas import tpu as pltpu
from jax.experimental.pallas import tpu_sc as plsc

EMBED_NUM = 200
EMBED_DIM = 64
BATCH = 16384
HIST = 200

_B = BATCH * HIST
_IDX_MINOR = 128
_IDX_ROWS = _B // _IDX_MINOR

_NW = 32
_ROWS_PER_W = _IDX_ROWS // _NW
_ROWS_PER_STEP = 8
_CHUNK = _ROWS_PER_STEP * _IDX_MINOR
_STEPS = _ROWS_PER_W // _ROWS_PER_STEP


def _sc_gather(idx2d, table):
    mesh = plsc.VectorSubcoreMesh(core_axis_name="c", subcore_axis_name="s")

    @functools.partial(
        pl.kernel,
        mesh=mesh,
        out_type=jax.ShapeDtypeStruct((_B, EMBED_DIM), jnp.float32),
        scratch_types=[
            pltpu.VMEM((_ROWS_PER_STEP, _IDX_MINOR), jnp.int32),
            pltpu.VMEM((_CHUNK, EMBED_DIM), jnp.float32),
            pltpu.VMEM_SHARED((EMBED_NUM, EMBED_DIM), jnp.float32),
            pltpu.SemaphoreType.DMA,
        ],
        compiler_params=pltpu.CompilerParams(use_tc_tiling_on_sc=False),
    )
    def k(idx_hbm, table_hbm, out_hbm, idx_v, rows_v, table_sp, sem):
        wid = lax.axis_index("s") * 2 + lax.axis_index("c")
        row0 = wid * _ROWS_PER_W

        @pl.when(lax.axis_index("s") == 0)
        def _():
            pltpu.sync_copy(table_hbm, table_sp)

        plsc.subcore_barrier()

        def step(i, _):
            r = row0 + i * _ROWS_PER_STEP
            pltpu.sync_copy(idx_hbm.at[pl.ds(r, _ROWS_PER_STEP)], idx_v)
            for j in range(_ROWS_PER_STEP):
                pltpu.async_copy(
                    table_sp.at[idx_v.at[j]],
                    rows_v.at[pl.ds(j * _IDX_MINOR, _IDX_MINOR)],
                    sem,
                )
            for j in range(_ROWS_PER_STEP):
                pltpu.make_async_copy(
                    table_sp.at[idx_v.at[j]],
                    rows_v.at[pl.ds(j * _IDX_MINOR, _IDX_MINOR)],
                    sem,
                ).wait()
            pltpu.sync_copy(rows_v, out_hbm.at[pl.ds(r * _IDX_MINOR, _CHUNK)])
            return ()

        lax.fori_loop(0, _STEPS, step, (), unroll=False)

    return k(idx2d, table)


def kernel(visit_order, pos_embed_weight):
    idx2d = jnp.reshape(visit_order.astype(jnp.int32), (_IDX_ROWS, _IDX_MINOR))
    flat = _sc_gather(idx2d, pos_embed_weight)
    return jnp.reshape(flat, (BATCH, HIST, EMBED_DIM))


# speedup vs baseline: 9.3721x; 1.7889x over previous
import functools

import jax
import jax.numpy as jnp
from jax import lax
from jax.experimental import pallas as pl
from jax.experimental.pallas import tpu as pltpu
from jax.experimental.pallas import tpu_sc as plsc

EMBED_NUM = 200
EMBED_DIM = 64
BATCH = 16384
HIST = 200

_B = BATCH * HIST
_IDX_MINOR = 128
_IDX_ROWS = _B // _IDX_MINOR

_NW = 32
_ROWS_PER_W = _IDX_ROWS // _NW
_ROWS_PER_CH = 2
_CHUNK = _ROWS_PER_CH * _IDX_MINOR
_NCH = _ROWS_PER_W // _ROWS_PER_CH
_NB = 2
_ITERS = _NCH // _NB


def _sc_gather(idx2d, table):
    mesh = plsc.VectorSubcoreMesh(core_axis_name="c", subcore_axis_name="s")

    @functools.partial(
        pl.kernel,
        mesh=mesh,
        out_type=jax.ShapeDtypeStruct((_B, EMBED_DIM), jnp.float32),
        scratch_types=[
            [pltpu.VMEM((_ROWS_PER_CH, _IDX_MINOR), jnp.int32)] * _NB,
            [pltpu.VMEM((_CHUNK, EMBED_DIM), jnp.float32)] * _NB,
            pltpu.VMEM_SHARED((EMBED_NUM, EMBED_DIM), jnp.float32),
            [pltpu.SemaphoreType.DMA] * _NB,
            [pltpu.SemaphoreType.DMA] * _NB,
        ],
    )
    def k(idx_hbm, table_hbm, out_hbm, idx_v, rows_v, table_sp, gsem, ssem):
        wid = lax.axis_index("s") * 2 + lax.axis_index("c")
        row0 = wid * _ROWS_PER_W

        @pl.when(lax.axis_index("s") == 0)
        def _():
            pltpu.sync_copy(table_hbm, table_sp)

        plsc.subcore_barrier()

        def rows_buf(b):
            return rows_v[b]

        def fire(b, c):
            r = row0 + c * _ROWS_PER_CH
            pltpu.sync_copy(idx_hbm.at[pl.ds(r, _ROWS_PER_CH)], idx_v[b])
            for j in range(_ROWS_PER_CH):
                pltpu.async_copy(
                    table_sp.at[idx_v[b].at[j]],
                    rows_v[b].at[pl.ds(j * _IDX_MINOR, _IDX_MINOR)],
                    gsem[b],
                )

        def wait_gather(b, c):
            for j in range(_ROWS_PER_CH):
                pltpu.make_async_copy(
                    table_sp.at[idx_v[b].at[j]],
                    rows_v[b].at[pl.ds(j * _IDX_MINOR, _IDX_MINOR)],
                    gsem[b],
                ).wait()

        def out_slice(c):
            return out_hbm.at[pl.ds((row0 + c * _ROWS_PER_CH) * _IDX_MINOR, _CHUNK)]

        for b in range(_NB):
            fire(b, b)

        def body(i, _):
            for b in range(_NB):
                c = i * _NB + b
                wait_gather(b, c)
                pltpu.async_copy(rows_buf(b), out_slice(c), ssem[b])
            for b in range(_NB):
                c = i * _NB + b
                pltpu.make_async_copy(rows_buf(b), out_slice(c), ssem[b]).wait()

                @pl.when(c + _NB < _NCH)
                def _():
                    fire(b, c + _NB)

            return ()

        lax.fori_loop(0, _ITERS, body, (), unroll=False)

    return k(idx2d, table)


def kernel(visit_order, pos_embed_weight):
    idx2d = jnp.reshape(visit_order.astype(jnp.int32), (_IDX_ROWS, _IDX_MINOR))
    flat = _sc_gather(idx2d, pos_embed_weight)
    return jnp.reshape(flat, (BATCH, HIST, EMBED_DIM))
